# baseline (speedup 1.0000x reference)
import functools

import jax
import jax.numpy as jnp
from jax import lax
from jax.experimental import pallas as pl
from jax.experimental.pallas import tpu as pltpu
from jax.experimental.pallas import tpu_sc as plsc

_NC = 2
_NS = 16
_NW = _NC * _NS
_LANES = 16
_CHUNK = 128
_BLK = 128
_SUB = 8
_CW = 64
_NCH = 4
_CAP = 1216


def _make_extract_call(B, NU, D):
    nblk_full = NU // _BLK
    per_w = -(-nblk_full // _NW)
    assert per_w >= _CW
    assert _NCH * _CW >= per_w
    ngrp_all = B // _LANES

    mesh = plsc.VectorSubcoreMesh(core_axis_name="c", subcore_axis_name="s")

    @functools.partial(
        pl.kernel,
        out_type=jax.ShapeDtypeStruct((B * D,), jnp.float32),
        mesh=mesh,
        compiler_params=pltpu.CompilerParams(
            needs_layout_passes=False, use_tc_tiling_on_sc=True),
        scratch_types=[
            pltpu.VMEM((B,), jnp.int32),
            pltpu.VMEM((_CAP + 16,), jnp.int32),
            pltpu.VMEM((_CAP + 16,), jnp.int32),
            pltpu.VMEM((B // _LANES + _LANES,), jnp.int32),
            pltpu.VMEM((_SUB, _CW * _BLK), jnp.float32),
            pltpu.VMEM((_CAP * D,), jnp.float32),
            pltpu.SemaphoreType.DMA,
        ],
    )
    def extract(ut_hbm, uidx_hbm, out_hbm,
                uall_v, me_v, mu_v, offs_v, sbuf, rows_v, sem):
        wid = lax.axis_index("s") * _NC + lax.axis_index("c")
        lo_blk = wid * per_w
        hi_blk = jnp.minimum(lo_blk + per_w, nblk_full)
        lo = lo_blk * _BLK
        hi = hi_blk * _BLK

        pltpu.sync_copy(uidx_hbm, uall_v)

        iota = lax.iota(jnp.int32, _LANES)

        lane0 = iota == 0

        def count(g, carry):
            u16 = uall_v[pl.ds(g * _LANES, _LANES)]
            m = (u16 >= lo) & (u16 < hi)
            c = plsc.all_reduce_population_count(m)
            plsc.store_scatter(offs_v, [jnp.full((_LANES,), g, jnp.int32)],
                               c, mask=lane0)
            return carry

        lax.fori_loop(0, ngrp_all, count, 0)

        def prefix(gg, carry):
            sl = pl.ds(gg * _LANES, _LANES)
            c16 = offs_v[sl]
            s = plsc.cumsum(c16) + carry
            offs_v[sl] = s - c16
            return s[_LANES - 1]

        cnt = lax.fori_loop(0, ngrp_all // _LANES, prefix, 0)
        ngrp = (cnt + _LANES - 1) // _LANES

        def scan(g, carry):
            u16 = uall_v[pl.ds(g * _LANES, _LANES)]
            m = (u16 >= lo) & (u16 < hi)
            ov = offs_v[pl.ds(g, _LANES)]
            off = ov[0]
            plsc.store_compressed(me_v.at[pl.ds(off, _LANES)],
                                  g * _LANES + iota, mask=m)
            plsc.store_compressed(mu_v.at[pl.ds(off, _LANES)], u16, mask=m)
            return carry

        lax.fori_loop(0, ngrp_all, scan, 0)

        for b in range(D // _SUB):
            def chunk_body(c, carry):
                sblk = jnp.minimum(lo_blk + c * _CW, hi_blk - _CW)
                start = pl.multiple_of(sblk * _BLK, _BLK)
                pltpu.async_copy(
                    ut_hbm.at[pl.ds(b * _SUB, _SUB),
                              pl.ds(start, _CW * _BLK)],
                    sbuf, sem).wait()

                def extr(k, carry2):
                    u16 = mu_v[pl.ds(k * _LANES, _LANES)]
                    m = (u16 >= start) & (u16 < start + _CW * _BLK)
                    uloc = u16 - start
                    slots = k * _LANES + iota
                    for drem in range(_SUB):
                        val = plsc.load_gather(
                            sbuf, [jnp.full((_LANES,), drem, jnp.int32),
                                   uloc], mask=m)
                        plsc.store_scatter(
                            rows_v,
                            [slots * D + (b * _SUB + drem)], val, mask=m)
                    return carry2

                lax.fori_loop(0, ngrp, extr, 0)
                return carry

            lax.fori_loop(0, _NCH, chunk_body, 0)

        def put(s, carry):
            ev = me_v[pl.ds(s, _LANES)]
            e = ev[0]
            pltpu.async_copy(rows_v.at[pl.ds(s * D, D)],
                             out_hbm.at[pl.ds(e * D, D)], sem)
            return carry

        lax.fori_loop(0, cnt, put, 0)

        def drain(s, carry):
            pltpu.make_async_copy(out_hbm.at[pl.ds(0, D)],
                                  rows_v.at[pl.ds(0, D)], sem).wait()
            return carry

        lax.fori_loop(0, cnt, drain, 0)

    return extract


def _make_compute_call(B, D, tail_base, tail_n):
    bpw = B // _NW
    assert bpw % _CHUNK == 0

    mesh = plsc.VectorSubcoreMesh(core_axis_name="c", subcore_axis_name="s")

    @functools.partial(
        pl.kernel,
        out_type=[
            jax.ShapeDtypeStruct((B,), jnp.float32),
            jax.ShapeDtypeStruct((B,), jnp.float32),
        ],
        mesh=mesh,
        compiler_params=pltpu.CompilerParams(
            needs_layout_passes=False, use_tc_tiling_on_sc=False),
        scratch_types=[
            pltpu.VMEM((bpw + _LANES,), jnp.int32),
            pltpu.VMEM((bpw,), jnp.int32),
            pltpu.VMEM((bpw * D,), jnp.float32),
            pltpu.VMEM((tail_n, D), jnp.float32),
            pltpu.VMEM((bpw, D), jnp.float32),
            pltpu.VMEM((bpw,), jnp.float32),
            pltpu.VMEM((bpw,), jnp.float32),
            pltpu.VMEM((_LANES,), jnp.float32),
            pltpu.VMEM((bpw,), jnp.float32),
            pltpu.VMEM((bpw,), jnp.float32),
            pltpu.SemaphoreType.DMA,
        ],
    )
    def compute(ui_hbm, utail_hbm, v_hbm, uidx_hbm, iidx_hbm, alpha_hbm,
                gu_hbm, gv_hbm, mu_hbm, sig_hbm,
                uidx_v, iidx_v, uflat_v, utail_v, v_rows,
                gu_v, gv_v, alpha_v, mu_v, sig_v, sem):
        wid = lax.axis_index("s") * _NC + lax.axis_index("c")
        base = wid * bpw

        pltpu.sync_copy(uidx_hbm.at[pl.ds(base, bpw)],
                        uidx_v.at[pl.ds(0, bpw)])
        pltpu.sync_copy(iidx_hbm.at[pl.ds(base, bpw)], iidx_v)
        pltpu.sync_copy(alpha_hbm, alpha_v)
        pltpu.sync_copy(utail_hbm, utail_v)
        pltpu.sync_copy(ui_hbm.at[pl.ds(base * D, bpw * D)], uflat_v)

        copies = []
        for j in range(bpw // _CHUNK):
            sl = pl.ds(j * _CHUNK, _CHUNK)
            copies.append(
                pltpu.async_copy(v_hbm.at[iidx_v.at[sl]], v_rows.at[sl],
                                 sem))
            copies.append(
                pltpu.async_copy(gu_hbm.at[uidx_v.at[sl]], gu_v.at[sl],
                                 sem))
            copies.append(
                pltpu.async_copy(gv_hbm.at[iidx_v.at[sl]], gv_v.at[sl],
                                 sem))
        for c in copies:
            c.wait()

        iota = lax.iota(jnp.int32, _LANES)
        lane0 = iota == 0

        def dot(e, carry):
            uvec = uidx_v[pl.ds(e, _LANES)]
            uid = uvec[0]
            is_tail = uid >= tail_base
            tr = jnp.where(is_tail, uid - tail_base, 0)
            tmask = jnp.full((_LANES,), is_tail)
            ua = jnp.where(tmask, utail_v[tr, pl.ds(0, _LANES)],
                           uflat_v[pl.ds(e * D, _LANES)])
            ub = jnp.where(tmask, utail_v[tr, pl.ds(_LANES, _LANES)],
                           uflat_v[pl.ds(e * D + _LANES, _LANES)])
            va = v_rows[e, pl.ds(0, _LANES)]
            vb = v_rows[e, pl.ds(_LANES, _LANES)]
            s = plsc.cumsum(ua * va + ub * vb)
            plsc.store_scatter(
                mu_v, [jnp.full((_LANES,), e, jnp.int32)],
                jnp.full((_LANES,), s[_LANES - 1]), mask=lane0)
            return carry

        lax.fori_loop(0, bpw, dot, 0)

        alpha = alpha_v[...]

        def sbody(g, carry):
            sl = pl.ds(g * _LANES, _LANES)
            x = alpha * gu_v[sl] * gv_v[sl]
            i = plsc.bitcast(x, jnp.int32)
            i = 0x5F3759DF - lax.shift_right_logical(i, 1)
            y = plsc.bitcast(i, jnp.float32)
            for _ in range(3):
                y = y * (1.5 - 0.5 * x * y * y)
            sig_v[sl] = y
            return carry

        lax.fori_loop(0, bpw // _LANES, sbody, 0)

        pltpu.sync_copy(mu_v, mu_hbm.at[pl.ds(base, bpw)])
        pltpu.sync_copy(sig_v, sig_hbm.at[pl.ds(base, bpw)])

    return compute


def kernel(user_idx, item_idx, U, V, alpha, gamma_u, gamma_v):
    B = user_idx.shape[0]
    NU, D = U.shape
    tail_base = (NU // _BLK) * _BLK
    tail_n = NU - tail_base
    assert D == 2 * _LANES and tail_n > 0

    uidx = user_idx.astype(jnp.int32)
    iidx = item_idx.astype(jnp.int32)
    alpha16 = jnp.broadcast_to(
        jnp.asarray(alpha, jnp.float32).reshape(()), (_LANES,))
    u_tail = U[tail_base:]

    ui_flat = _make_extract_call(B, NU, D)(U.T, uidx)
    mu, sigma = _make_compute_call(B, D, tail_base, tail_n)(
        ui_flat, u_tail, V, uidx, iidx, alpha16, gamma_u, gamma_v)
    return (mu, sigma)


# speedup vs baseline: 1.0583x; 1.0583x over previous
import functools

import jax
import jax.numpy as jnp
from jax import lax
from jax.experimental import pallas as pl
from jax.experimental.pallas import tpu as pltpu
from jax.experimental.pallas import tpu_sc as plsc

_NC = 2
_NS = 16
_NW = _NC * _NS
_LANES = 16
_CHUNK = 128
_BLK = 128
_SUB = 8
_CW = 64
_NCH = 4
_CAP = 1216


def _make_extract_call(B, NU, D):
    nblk_full = NU // _BLK
    per_w = -(-nblk_full // _NW)
    assert per_w >= _CW
    assert _NCH * _CW >= per_w
    ngrp_all = B // _LANES

    mesh = plsc.VectorSubcoreMesh(core_axis_name="c", subcore_axis_name="s")

    @functools.partial(
        pl.kernel,
        out_type=jax.ShapeDtypeStruct((B * D,), jnp.float32),
        mesh=mesh,
        compiler_params=pltpu.CompilerParams(
            needs_layout_passes=False, use_tc_tiling_on_sc=True),
        scratch_types=[
            pltpu.VMEM((B,), jnp.int32),
            pltpu.VMEM((_CAP + 16,), jnp.int32),
            pltpu.VMEM((_CAP + 16,), jnp.int32),
            pltpu.VMEM((_SUB, _CW * _BLK), jnp.float32),
            pltpu.VMEM((_CAP * D,), jnp.float32),
            pltpu.SemaphoreType.DMA,
        ],
    )
    def extract(ut_hbm, uidx_hbm, out_hbm,
                uall_v, me_v, mu_v, sbuf, rows_v, sem):
        wid = lax.axis_index("s") * _NC + lax.axis_index("c")
        lo_blk = wid * per_w
        hi_blk = jnp.minimum(lo_blk + per_w, nblk_full)
        lo = lo_blk * _BLK
        hi = hi_blk * _BLK

        pltpu.sync_copy(uidx_hbm, uall_v)

        iota = lax.iota(jnp.int32, _LANES)

        def scan(g, cnt):
            u16 = uall_v[pl.ds(g * _LANES, _LANES)]
            m = (u16 >= lo) & (u16 < hi)
            c = plsc.all_reduce_population_count(m)
            plsc.store_compressed(me_v.at[pl.ds(cnt, _LANES)],
                                  g * _LANES + iota, mask=m)
            plsc.store_compressed(mu_v.at[pl.ds(cnt, _LANES)], u16, mask=m)
            return cnt + c[0]

        cnt = lax.fori_loop(0, ngrp_all, scan, 0)
        ngrp = (cnt + _LANES - 1) // _LANES

        for b in range(D // _SUB):
            def chunk_body(c, carry):
                sblk = jnp.minimum(lo_blk + c * _CW, hi_blk - _CW)
                start = pl.multiple_of(sblk * _BLK, _BLK)
                pltpu.async_copy(
                    ut_hbm.at[pl.ds(b * _SUB, _SUB),
                              pl.ds(start, _CW * _BLK)],
                    sbuf, sem).wait()

                def extr(k, carry2):
                    u16 = mu_v[pl.ds(k * _LANES, _LANES)]
                    m = (u16 >= start) & (u16 < start + _CW * _BLK)
                    uloc = u16 - start
                    slots = k * _LANES + iota
                    for drem in range(_SUB):
                        val = plsc.load_gather(
                            sbuf, [jnp.full((_LANES,), drem, jnp.int32),
                                   uloc], mask=m)
                        plsc.store_scatter(
                            rows_v,
                            [slots * D + (b * _SUB + drem)], val, mask=m)
                    return carry2

                lax.fori_loop(0, ngrp, extr, 0)
                return carry

            lax.fori_loop(0, _NCH, chunk_body, 0)

        def put(s, carry):
            ev = me_v[pl.ds(s, _LANES)]
            e = ev[0]
            pltpu.async_copy(rows_v.at[pl.ds(s * D, D)],
                             out_hbm.at[pl.ds(e * D, D)], sem)
            return carry

        lax.fori_loop(0, cnt, put, 0)

        def drain(s, carry):
            pltpu.make_async_copy(out_hbm.at[pl.ds(0, D)],
                                  rows_v.at[pl.ds(0, D)], sem).wait()
            return carry

        lax.fori_loop(0, cnt, drain, 0)

    return extract


def _make_compute_call(B, D, tail_base, tail_n):
    bpw = B // _NW
    assert bpw % _CHUNK == 0

    mesh = plsc.VectorSubcoreMesh(core_axis_name="c", subcore_axis_name="s")

    @functools.partial(
        pl.kernel,
        out_type=[
            jax.ShapeDtypeStruct((B,), jnp.float32),
            jax.ShapeDtypeStruct((B,), jnp.float32),
        ],
        mesh=mesh,
        compiler_params=pltpu.CompilerParams(
            needs_layout_passes=False, use_tc_tiling_on_sc=False),
        scratch_types=[
            pltpu.VMEM((bpw + _LANES,), jnp.int32),
            pltpu.VMEM((bpw,), jnp.int32),
            pltpu.VMEM((bpw * D,), jnp.float32),
            pltpu.VMEM((tail_n, D), jnp.float32),
            pltpu.VMEM((bpw, D), jnp.float32),
            pltpu.VMEM((bpw,), jnp.float32),
            pltpu.VMEM((bpw,), jnp.float32),
            pltpu.VMEM((_LANES,), jnp.float32),
            pltpu.VMEM((bpw,), jnp.float32),
            pltpu.VMEM((bpw,), jnp.float32),
            pltpu.SemaphoreType.DMA,
        ],
    )
    def compute(ui_hbm, utail_hbm, v_hbm, uidx_hbm, iidx_hbm, alpha_hbm,
                gu_hbm, gv_hbm, mu_hbm, sig_hbm,
                uidx_v, iidx_v, uflat_v, utail_v, v_rows,
                gu_v, gv_v, alpha_v, mu_v, sig_v, sem):
        wid = lax.axis_index("s") * _NC + lax.axis_index("c")
        base = wid * bpw

        pltpu.sync_copy(uidx_hbm.at[pl.ds(base, bpw)],
                        uidx_v.at[pl.ds(0, bpw)])
        pltpu.sync_copy(iidx_hbm.at[pl.ds(base, bpw)], iidx_v)
        pltpu.sync_copy(alpha_hbm, alpha_v)
        pltpu.sync_copy(utail_hbm, utail_v)
        pltpu.sync_copy(ui_hbm.at[pl.ds(base * D, bpw * D)], uflat_v)

        copies = []
        for j in range(bpw // _CHUNK):
            sl = pl.ds(j * _CHUNK, _CHUNK)
            copies.append(
                pltpu.async_copy(v_hbm.at[iidx_v.at[sl]], v_rows.at[sl],
                                 sem))
            copies.append(
                pltpu.async_copy(gu_hbm.at[uidx_v.at[sl]], gu_v.at[sl],
                                 sem))
            copies.append(
                pltpu.async_copy(gv_hbm.at[iidx_v.at[sl]], gv_v.at[sl],
                                 sem))
        for c in copies:
            c.wait()

        iota = lax.iota(jnp.int32, _LANES)
        lane0 = iota == 0

        def dot(e, carry):
            uvec = uidx_v[pl.ds(e, _LANES)]
            uid = uvec[0]
            is_tail = uid >= tail_base
            tr = jnp.where(is_tail, uid - tail_base, 0)
            tmask = jnp.full((_LANES,), is_tail)
            ua = jnp.where(tmask, utail_v[tr, pl.ds(0, _LANES)],
                           uflat_v[pl.ds(e * D, _LANES)])
            ub = jnp.where(tmask, utail_v[tr, pl.ds(_LANES, _LANES)],
                           uflat_v[pl.ds(e * D + _LANES, _LANES)])
            va = v_rows[e, pl.ds(0, _LANES)]
            vb = v_rows[e, pl.ds(_LANES, _LANES)]
            s = plsc.cumsum(ua * va + ub * vb)
            plsc.store_scatter(
                mu_v, [jnp.full((_LANES,), e, jnp.int32)],
                jnp.full((_LANES,), s[_LANES - 1]), mask=lane0)
            return carry

        lax.fori_loop(0, bpw, dot, 0)

        alpha = alpha_v[...]

        def sbody(g, carry):
            sl = pl.ds(g * _LANES, _LANES)
            x = alpha * gu_v[sl] * gv_v[sl]
            i = plsc.bitcast(x, jnp.int32)
            i = 0x5F3759DF - lax.shift_right_logical(i, 1)
            y = plsc.bitcast(i, jnp.float32)
            for _ in range(3):
                y = y * (1.5 - 0.5 * x * y * y)
            sig_v[sl] = y
            return carry

        lax.fori_loop(0, bpw // _LANES, sbody, 0)

        pltpu.sync_copy(mu_v, mu_hbm.at[pl.ds(base, bpw)])
        pltpu.sync_copy(sig_v, sig_hbm.at[pl.ds(base, bpw)])

    return compute


def kernel(user_idx, item_idx, U, V, alpha, gamma_u, gamma_v):
    B = user_idx.shape[0]
    NU, D = U.shape
    tail_base = (NU // _BLK) * _BLK
    tail_n = NU - tail_base
    assert D == 2 * _LANES and tail_n > 0

    uidx = user_idx.astype(jnp.int32)
    iidx = item_idx.astype(jnp.int32)
    alpha16 = jnp.broadcast_to(
        jnp.asarray(alpha, jnp.float32).reshape(()), (_LANES,))
    u_tail = U[tail_base:]

    ui_flat = _make_extract_call(B, NU, D)(U.T, uidx)
    mu, sigma = _make_compute_call(B, D, tail_base, tail_n)(
        ui_flat, u_tail, V, uidx, iidx, alpha16, gamma_u, gamma_v)
    return (mu, sigma)


# speedup vs baseline: 1.0637x; 1.0051x over previous
import functools

import jax
import jax.numpy as jnp
from jax import lax
from jax.experimental import pallas as pl
from jax.experimental.pallas import tpu as pltpu
from jax.experimental.pallas import tpu_sc as plsc

_NC = 2
_NS = 16
_NW = _NC * _NS
_LANES = 16
_CHUNK = 128
_BLK = 128
_SUB = 8
_CW = 64
_NCH = 4
_CAP = 1216


def _make_extract_call(B, NU, D):
    nblk_full = NU // _BLK
    per_w = -(-nblk_full // _NW)
    assert per_w >= _CW
    assert _NCH * _CW >= per_w
    ngrp_all = B // _LANES

    mesh = plsc.VectorSubcoreMesh(core_axis_name="c", subcore_axis_name="s")

    @functools.partial(
        pl.kernel,
        out_type=jax.ShapeDtypeStruct((B * D,), jnp.float32),
        mesh=mesh,
        compiler_params=pltpu.CompilerParams(
            needs_layout_passes=False, use_tc_tiling_on_sc=True),
        scratch_types=[
            pltpu.VMEM((B,), jnp.int32),
            pltpu.VMEM((_CAP + 16,), jnp.int32),
            pltpu.VMEM((_CAP + 16,), jnp.int32),
            pltpu.VMEM((_SUB, _CW * _BLK), jnp.float32),
            pltpu.VMEM((_CAP * D,), jnp.float32),
            pltpu.SemaphoreType.DMA,
        ],
    )
    def extract(ut_hbm, uidx_hbm, out_hbm,
                uall_v, me_v, mu_v, sbuf, rows_v, sem):
        wid = lax.axis_index("s") * _NC + lax.axis_index("c")
        lo_blk = wid * per_w
        hi_blk = jnp.minimum(lo_blk + per_w, nblk_full)
        lo = lo_blk * _BLK
        hi = hi_blk * _BLK

        pltpu.sync_copy(uidx_hbm, uall_v)

        iota = lax.iota(jnp.int32, _LANES)

        def scan(g, cnt):
            u16 = uall_v[pl.ds(g * _LANES, _LANES)]
            m = (u16 >= lo) & (u16 < hi)
            c = plsc.all_reduce_population_count(m)
            plsc.store_compressed(me_v.at[pl.ds(cnt, _LANES)],
                                  g * _LANES + iota, mask=m)
            plsc.store_compressed(mu_v.at[pl.ds(cnt, _LANES)], u16, mask=m)
            return cnt + c[0]

        cnt = lax.fori_loop(0, ngrp_all, scan, 0)
        ngrp = (cnt + _LANES - 1) // _LANES

        for b in range(D // _SUB):
            def chunk_body(c, carry):
                sblk = jnp.minimum(lo_blk + c * _CW, hi_blk - _CW)
                start = pl.multiple_of(sblk * _BLK, _BLK)
                pltpu.async_copy(
                    ut_hbm.at[pl.ds(b * _SUB, _SUB),
                              pl.ds(start, _CW * _BLK)],
                    sbuf, sem).wait()

                def extr(k, carry2):
                    u16 = mu_v[pl.ds(k * _LANES, _LANES)]
                    m = (u16 >= start) & (u16 < start + _CW * _BLK)
                    uloc = u16 - start
                    slots = k * _LANES + iota
                    for drem in range(_SUB):
                        val = plsc.load_gather(
                            sbuf, [jnp.full((_LANES,), drem, jnp.int32),
                                   uloc], mask=m)
                        plsc.store_scatter(
                            rows_v,
                            [slots * D + (b * _SUB + drem)], val, mask=m)
                    return carry2

                lax.fori_loop(0, ngrp, extr, 0)
                return carry

            lax.fori_loop(0, _NCH, chunk_body, 0)

        def put(s, carry):
            ev = me_v[pl.ds(s, _LANES)]
            e = ev[0]
            pltpu.async_copy(rows_v.at[pl.ds(s * D, D)],
                             out_hbm.at[pl.ds(e * D, D)], sem)
            return carry

        lax.fori_loop(0, cnt, put, 0)

        def drain(s, carry):
            pltpu.make_async_copy(out_hbm.at[pl.ds(0, D)],
                                  rows_v.at[pl.ds(0, D)], sem).wait()
            return carry

        lax.fori_loop(0, cnt, drain, 0)

    return extract


def _make_compute_call(B, D, tail_base, tail_n):
    bpw = B // _NW
    assert bpw % _CHUNK == 0

    mesh = plsc.VectorSubcoreMesh(core_axis_name="c", subcore_axis_name="s")

    @functools.partial(
        pl.kernel,
        out_type=[
            jax.ShapeDtypeStruct((B,), jnp.float32),
            jax.ShapeDtypeStruct((B,), jnp.float32),
        ],
        mesh=mesh,
        compiler_params=pltpu.CompilerParams(
            needs_layout_passes=False, use_tc_tiling_on_sc=False),
        scratch_types=[
            pltpu.VMEM((bpw + _LANES,), jnp.int32),
            pltpu.VMEM((bpw,), jnp.int32),
            pltpu.VMEM((bpw * D,), jnp.float32),
            pltpu.VMEM((tail_n, D), jnp.float32),
            pltpu.VMEM((bpw, D), jnp.float32),
            pltpu.VMEM((bpw,), jnp.float32),
            pltpu.VMEM((bpw,), jnp.float32),
            pltpu.VMEM((_LANES,), jnp.float32),
            pltpu.VMEM((bpw,), jnp.float32),
            pltpu.VMEM((bpw,), jnp.float32),
            pltpu.SemaphoreType.DMA,
        ],
    )
    def compute(ui_hbm, utail_hbm, v_hbm, uidx_hbm, iidx_hbm, alpha_hbm,
                gu_hbm, gv_hbm, mu_hbm, sig_hbm,
                uidx_v, iidx_v, uflat_v, utail_v, v_rows,
                gu_v, gv_v, alpha_v, mu_v, sig_v, sem):
        wid = lax.axis_index("s") * _NC + lax.axis_index("c")
        base = wid * bpw

        pltpu.sync_copy(uidx_hbm.at[pl.ds(base, bpw)],
                        uidx_v.at[pl.ds(0, bpw)])
        pltpu.sync_copy(iidx_hbm.at[pl.ds(base, bpw)], iidx_v)
        pltpu.sync_copy(alpha_hbm, alpha_v)
        pltpu.sync_copy(utail_hbm, utail_v)
        pltpu.sync_copy(ui_hbm.at[pl.ds(base * D, bpw * D)], uflat_v)

        copies = []
        for j in range(bpw // _CHUNK):
            sl = pl.ds(j * _CHUNK, _CHUNK)
            copies.append(
                pltpu.async_copy(v_hbm.at[iidx_v.at[sl]], v_rows.at[sl],
                                 sem))
            copies.append(
                pltpu.async_copy(gu_hbm.at[uidx_v.at[sl]], gu_v.at[sl],
                                 sem))
            copies.append(
                pltpu.async_copy(gv_hbm.at[iidx_v.at[sl]], gv_v.at[sl],
                                 sem))
        for c in copies:
            c.wait()

        iota = lax.iota(jnp.int32, _LANES)

        def dot(g, carry):
            sl = pl.ds(g * _LANES, _LANES)
            rows = g * _LANES + iota
            uid16 = uidx_v[sl]
            mt = uid16 >= tail_base
            trow = jnp.where(mt, uid16 - tail_base, 0)
            ubase = rows * D
            col = jnp.zeros((_LANES,), jnp.int32)
            acc = jnp.zeros((_LANES,), jnp.float32)
            for _ in range(D):
                un = plsc.load_gather(uflat_v, [ubase + col])
                tn = plsc.load_gather(utail_v, [trow, col], mask=mt)
                un = jnp.where(mt, tn, un)
                vn = plsc.load_gather(v_rows, [rows, col])
                acc = acc + un * vn
                col = col + 1
            mu_v[sl] = acc
            return carry

        lax.fori_loop(0, bpw // _LANES, dot, 0)

        alpha = alpha_v[...]

        def sbody(g, carry):
            sl = pl.ds(g * _LANES, _LANES)
            x = alpha * gu_v[sl] * gv_v[sl]
            i = plsc.bitcast(x, jnp.int32)
            i = 0x5F3759DF - lax.shift_right_logical(i, 1)
            y = plsc.bitcast(i, jnp.float32)
            for _ in range(3):
                y = y * (1.5 - 0.5 * x * y * y)
            sig_v[sl] = y
            return carry

        lax.fori_loop(0, bpw // _LANES, sbody, 0)

        pltpu.sync_copy(mu_v, mu_hbm.at[pl.ds(base, bpw)])
        pltpu.sync_copy(sig_v, sig_hbm.at[pl.ds(base, bpw)])

    return compute


def kernel(user_idx, item_idx, U, V, alpha, gamma_u, gamma_v):
    B = user_idx.shape[0]
    NU, D = U.shape
    tail_base = (NU // _BLK) * _BLK
    tail_n = NU - tail_base
    assert D == 2 * _LANES and tail_n > 0

    uidx = user_idx.astype(jnp.int32)
    iidx = item_idx.astype(jnp.int32)
    alpha16 = jnp.broadcast_to(
        jnp.asarray(alpha, jnp.float32).reshape(()), (_LANES,))
    u_tail = U[tail_base:]

    ui_flat = _make_extract_call(B, NU, D)(U.T, uidx)
    mu, sigma = _make_compute_call(B, D, tail_base, tail_n)(
        ui_flat, u_tail, V, uidx, iidx, alpha16, gamma_u, gamma_v)
    return (mu, sigma)


# speedup vs baseline: 1.0833x; 1.0185x over previous
import functools

import jax
import jax.numpy as jnp
from jax import lax
from jax.experimental import pallas as pl
from jax.experimental.pallas import tpu as pltpu
from jax.experimental.pallas import tpu_sc as plsc

_NC = 2
_NS = 16
_NW = _NC * _NS
_LANES = 16
_CHUNK = 128
_BLK = 128
_SUB = 8
_CW = 32
_NCH = 8
_CAP = 1216


def _make_extract_call(B, NU, D):
    nblk_full = NU // _BLK
    per_w = -(-nblk_full // _NW)
    assert per_w >= _CW
    assert _NCH * _CW >= per_w
    ngrp_all = B // _LANES

    mesh = plsc.VectorSubcoreMesh(core_axis_name="c", subcore_axis_name="s")

    @functools.partial(
        pl.kernel,
        out_type=jax.ShapeDtypeStruct((B * D,), jnp.float32),
        mesh=mesh,
        compiler_params=pltpu.CompilerParams(
            needs_layout_passes=False, use_tc_tiling_on_sc=True),
        scratch_types=[
            pltpu.VMEM((B,), jnp.int32),
            pltpu.VMEM((_CAP + 16,), jnp.int32),
            pltpu.VMEM((_CAP + 16,), jnp.int32),
            pltpu.VMEM((_SUB, _CW * _BLK), jnp.float32),
            pltpu.VMEM((_SUB, _CW * _BLK), jnp.float32),
            pltpu.VMEM((_CAP * D,), jnp.float32),
            pltpu.SemaphoreType.DMA,
            pltpu.SemaphoreType.DMA,
            pltpu.SemaphoreType.DMA,
        ],
    )
    def extract(ut_hbm, uidx_hbm, out_hbm,
                uall_v, me_v, mu_v, sbufa, sbufb, rows_v,
                sem, sem_sa, sem_sb):
        wid = lax.axis_index("s") * _NC + lax.axis_index("c")
        lo_blk = wid * per_w
        hi_blk = jnp.minimum(lo_blk + per_w, nblk_full)
        lo = lo_blk * _BLK
        hi = hi_blk * _BLK

        pltpu.sync_copy(uidx_hbm, uall_v)

        iota = lax.iota(jnp.int32, _LANES)

        def scan(g, cnt):
            u16 = uall_v[pl.ds(g * _LANES, _LANES)]
            m = (u16 >= lo) & (u16 < hi)
            c = plsc.all_reduce_population_count(m)
            plsc.store_compressed(me_v.at[pl.ds(cnt, _LANES)],
                                  g * _LANES + iota, mask=m)
            plsc.store_compressed(mu_v.at[pl.ds(cnt, _LANES)], u16, mask=m)
            return cnt + c[0]

        cnt = lax.fori_loop(0, ngrp_all, scan, 0)
        ngrp = (cnt + _LANES - 1) // _LANES

        chunks = [(b, c) for b in range(D // _SUB) for c in range(_NCH)]
        bufs = (sbufa, sbufb)
        sems = (sem_sa, sem_sb)

        def issue(i):
            b, c = chunks[i]
            sblk = jnp.minimum(lo_blk + c * _CW, hi_blk - _CW)
            start = pl.multiple_of(sblk * _BLK, _BLK)
            cp = pltpu.async_copy(
                ut_hbm.at[pl.ds(b * _SUB, _SUB),
                          pl.ds(start, _CW * _BLK)],
                bufs[i % 2], sems[i % 2])
            return start, cp

        def process(start, buf, boff):
            def extr(k, carry2):
                u16 = mu_v[pl.ds(k * _LANES, _LANES)]
                m = (u16 >= start) & (u16 < start + _CW * _BLK)
                uloc = u16 - start
                slots = k * _LANES + iota
                for drem in range(_SUB):
                    val = plsc.load_gather(
                        buf, [jnp.full((_LANES,), drem, jnp.int32),
                              uloc], mask=m)
                    plsc.store_scatter(
                        rows_v, [slots * D + (boff + drem)], val, mask=m)
                return carry2

            lax.fori_loop(0, ngrp, extr, 0)

        pend = issue(0)
        for i in range(len(chunks)):
            start, cp = pend
            cp.wait()
            if i + 1 < len(chunks):
                pend = issue(i + 1)
            process(start, bufs[i % 2], chunks[i][0] * _SUB)

        def put(s, carry):
            ev = me_v[pl.ds(s, _LANES)]
            e = ev[0]
            pltpu.async_copy(rows_v.at[pl.ds(s * D, D)],
                             out_hbm.at[pl.ds(e * D, D)], sem)
            return carry

        lax.fori_loop(0, cnt, put, 0)

        def drain(s, carry):
            pltpu.make_async_copy(out_hbm.at[pl.ds(0, D)],
                                  rows_v.at[pl.ds(0, D)], sem).wait()
            return carry

        lax.fori_loop(0, cnt, drain, 0)

    return extract


def _make_compute_call(B, D, tail_base, tail_n):
    bpw = B // _NW
    assert bpw % _CHUNK == 0

    mesh = plsc.VectorSubcoreMesh(core_axis_name="c", subcore_axis_name="s")

    @functools.partial(
        pl.kernel,
        out_type=[
            jax.ShapeDtypeStruct((B,), jnp.float32),
            jax.ShapeDtypeStruct((B,), jnp.float32),
        ],
        mesh=mesh,
        compiler_params=pltpu.CompilerParams(
            needs_layout_passes=False, use_tc_tiling_on_sc=False),
        scratch_types=[
            pltpu.VMEM((bpw + _LANES,), jnp.int32),
            pltpu.VMEM((bpw,), jnp.int32),
            pltpu.VMEM((bpw * D,), jnp.float32),
            pltpu.VMEM((tail_n, D), jnp.float32),
            pltpu.VMEM((bpw, D), jnp.float32),
            pltpu.VMEM((bpw,), jnp.float32),
            pltpu.VMEM((bpw,), jnp.float32),
            pltpu.VMEM((_LANES,), jnp.float32),
            pltpu.VMEM((bpw,), jnp.float32),
            pltpu.VMEM((bpw,), jnp.float32),
            pltpu.SemaphoreType.DMA,
        ],
    )
    def compute(ui_hbm, utail_hbm, v_hbm, uidx_hbm, iidx_hbm, alpha_hbm,
                gu_hbm, gv_hbm, mu_hbm, sig_hbm,
                uidx_v, iidx_v, uflat_v, utail_v, v_rows,
                gu_v, gv_v, alpha_v, mu_v, sig_v, sem):
        wid = lax.axis_index("s") * _NC + lax.axis_index("c")
        base = wid * bpw

        pltpu.sync_copy(uidx_hbm.at[pl.ds(base, bpw)],
                        uidx_v.at[pl.ds(0, bpw)])
        pltpu.sync_copy(iidx_hbm.at[pl.ds(base, bpw)], iidx_v)
        pltpu.sync_copy(alpha_hbm, alpha_v)
        pltpu.sync_copy(utail_hbm, utail_v)
        pltpu.sync_copy(ui_hbm.at[pl.ds(base * D, bpw * D)], uflat_v)

        copies = []
        for j in range(bpw // _CHUNK):
            sl = pl.ds(j * _CHUNK, _CHUNK)
            copies.append(
                pltpu.async_copy(v_hbm.at[iidx_v.at[sl]], v_rows.at[sl],
                                 sem))
            copies.append(
                pltpu.async_copy(gu_hbm.at[uidx_v.at[sl]], gu_v.at[sl],
                                 sem))
            copies.append(
                pltpu.async_copy(gv_hbm.at[iidx_v.at[sl]], gv_v.at[sl],
                                 sem))
        for c in copies:
            c.wait()

        iota = lax.iota(jnp.int32, _LANES)

        def dot(g, carry):
            sl = pl.ds(g * _LANES, _LANES)
            rows = g * _LANES + iota
            uid16 = uidx_v[sl]
            mt = uid16 >= tail_base
            trow = jnp.where(mt, uid16 - tail_base, 0)
            ubase = rows * D
            col = jnp.zeros((_LANES,), jnp.int32)
            acc = jnp.zeros((_LANES,), jnp.float32)
            for _ in range(D):
                un = plsc.load_gather(uflat_v, [ubase + col])
                tn = plsc.load_gather(utail_v, [trow, col], mask=mt)
                un = jnp.where(mt, tn, un)
                vn = plsc.load_gather(v_rows, [rows, col])
                acc = acc + un * vn
                col = col + 1
            mu_v[sl] = acc
            return carry

        lax.fori_loop(0, bpw // _LANES, dot, 0)

        alpha = alpha_v[...]

        def sbody(g, carry):
            sl = pl.ds(g * _LANES, _LANES)
            x = alpha * gu_v[sl] * gv_v[sl]
            i = plsc.bitcast(x, jnp.int32)
            i = 0x5F3759DF - lax.shift_right_logical(i, 1)
            y = plsc.bitcast(i, jnp.float32)
            for _ in range(3):
                y = y * (1.5 - 0.5 * x * y * y)
            sig_v[sl] = y
            return carry

        lax.fori_loop(0, bpw // _LANES, sbody, 0)

        pltpu.sync_copy(mu_v, mu_hbm.at[pl.ds(base, bpw)])
        pltpu.sync_copy(sig_v, sig_hbm.at[pl.ds(base, bpw)])

    return compute


def kernel(user_idx, item_idx, U, V, alpha, gamma_u, gamma_v):
    B = user_idx.shape[0]
    NU, D = U.shape
    tail_base = (NU // _BLK) * _BLK
    tail_n = NU - tail_base
    assert D == 2 * _LANES and tail_n > 0

    uidx = user_idx.astype(jnp.int32)
    iidx = item_idx.astype(jnp.int32)
    alpha16 = jnp.broadcast_to(
        jnp.asarray(alpha, jnp.float32).reshape(()), (_LANES,))
    u_tail = U[tail_base:]

    ui_flat = _make_extract_call(B, NU, D)(U.T, uidx)
    mu, sigma = _make_compute_call(B, D, tail_base, tail_n)(
        ui_flat, u_tail, V, uidx, iidx, alpha16, gamma_u, gamma_v)
    return (mu, sigma)
